# baseline (speedup 1.0000x reference)
import jax
import jax.numpy as jnp
from jax.experimental import pallas as pl
from jax.experimental.pallas import tpu as pltpu


def _cummax_body(x_ref, o_ref, carry_ref):
    j = pl.program_id(1)
    y = x_ref[0, 0]
    hc, w = y.shape
    neg_inf = jnp.float32(-jnp.inf)

    s = 1
    while s < hc:
        pad = jnp.full((s, w), neg_inf, y.dtype)
        shifted = jnp.concatenate([pad, y[:-s]], axis=0)
        y = jnp.maximum(y, shifted)
        s *= 2

    @pl.when(j == 0)
    def _():
        carry_ref[...] = jnp.full(carry_ref.shape, neg_inf, y.dtype)

    y = jnp.maximum(y, carry_ref[7:8])
    o_ref[0, 0] = y
    carry_ref[...] = y[-8:]


def kernel(x):
    b, c, h, w = x.shape
    hc = 256 if h % 256 == 0 else h
    return pl.pallas_call(
        _cummax_body,
        grid=(b, h // hc),
        in_specs=[pl.BlockSpec((1, c, hc, w), lambda i, j: (i, 0, j, 0))],
        out_specs=pl.BlockSpec((1, c, hc, w), lambda i, j: (i, 0, j, 0)),
        out_shape=jax.ShapeDtypeStruct(x.shape, x.dtype),
        scratch_shapes=[pltpu.VMEM((8, w), jnp.float32)],
        compiler_params=pltpu.CompilerParams(
            dimension_semantics=("parallel", "arbitrary"),
        ),
    )(x)
